# baseline (speedup 1.0000x reference)
import functools

import jax
import jax.numpy as jnp
from jax import lax
from jax.experimental import pallas as pl
from jax.experimental.pallas import tpu as pltpu
from jax.experimental.pallas import tpu_sc as plsc

VOCAB = 1000000
EMBED_DIM = 64
UNITS = 128
BATCH = 1024
SEQ = 50

_NC = 2
_NS = 16
_NW = _NC * _NS
_ROWS = BATCH * SEQ
_RPW = _ROWS // _NW
_CW = 80
_CH = _RPW // _CW


@functools.lru_cache(maxsize=1)
def _make_sc_gather():
    mesh = plsc.VectorSubcoreMesh(core_axis_name="c", subcore_axis_name="s")

    @functools.partial(
        pl.kernel,
        mesh=mesh,
        out_type=jax.ShapeDtypeStruct((_NW, _RPW, EMBED_DIM), jnp.float32),
        scratch_types=[
            pltpu.VMEM((_CH, _CW), jnp.int32),
            pltpu.VMEM((_RPW, EMBED_DIM), jnp.float32),
            pltpu.SemaphoreType.DMA,
        ],
        compiler_params=pltpu.CompilerParams(use_tc_tiling_on_sc=False),
    )
    def sc_gather(table_hbm, idx_hbm, out_hbm, idx_v, rows_v, sem):
        wid = lax.axis_index("s") * _NC + lax.axis_index("c")
        pltpu.sync_copy(idx_hbm.at[wid], idx_v)
        copies = []
        for j in range(_CH):
            copies.append(
                pltpu.async_copy(
                    table_hbm.at[idx_v.at[j]],
                    rows_v.at[pl.ds(j * _CW, _CW)],
                    sem,
                )
            )
        for cp in copies:
            cp.wait()
        pltpu.sync_copy(rows_v, out_hbm.at[wid])

    return sc_gather


def _gru_body(emb_ref, h0_ref, wk_ref, wr_ref, bi_ref, br_ref, out_ref, h_ref):
    t = pl.program_id(0)

    @pl.when(t == 0)
    def _():
        h_ref[...] = h0_ref[...]

    h = h_ref[...]
    xt = emb_ref[0]
    matx = jnp.dot(xt, wk_ref[...], preferred_element_type=jnp.float32) + bi_ref[...]
    math = jnp.dot(h, wr_ref[...], preferred_element_type=jnp.float32) + br_ref[...]
    xz = matx[:, 0:UNITS]
    xr = matx[:, UNITS:2 * UNITS]
    xh = matx[:, 2 * UNITS:3 * UNITS]
    hz = math[:, 0:UNITS]
    hr = math[:, UNITS:2 * UNITS]
    hh_rec = math[:, 2 * UNITS:3 * UNITS]
    z = jax.nn.sigmoid(xz + hz)
    r = jax.nn.sigmoid(xr + hr)
    hh = jnp.tanh(xh + r * hh_rec)
    h_new = z * h + (1.0 - z) * hh
    h_ref[...] = h_new
    out_ref[0] = h_new


def _gru_scan(emb, h0, wk, wr, bi, br):
    return pl.pallas_call(
        _gru_body,
        grid=(SEQ,),
        in_specs=[
            pl.BlockSpec((1, BATCH, EMBED_DIM), lambda t: (t, 0, 0)),
            pl.BlockSpec((BATCH, UNITS), lambda t: (0, 0)),
            pl.BlockSpec((EMBED_DIM, 3 * UNITS), lambda t: (0, 0)),
            pl.BlockSpec((UNITS, 3 * UNITS), lambda t: (0, 0)),
            pl.BlockSpec((1, 3 * UNITS), lambda t: (0, 0)),
            pl.BlockSpec((1, 3 * UNITS), lambda t: (0, 0)),
        ],
        out_specs=pl.BlockSpec((1, BATCH, UNITS), lambda t: (t, 0, 0)),
        out_shape=jax.ShapeDtypeStruct((SEQ, BATCH, UNITS), jnp.float32),
        scratch_shapes=[pltpu.VMEM((BATCH, UNITS), jnp.float32)],
    )(emb, h0, wk, wr, bi, br)


def kernel(x, gru_init_state, embedding, kernel, recurrent_kernel, bias_input, bias_recurrent):
    idx = jnp.transpose(x.astype(jnp.int32), (1, 0)).reshape(_NW, _CH, _CW)
    rows = _make_sc_gather()(embedding, idx)
    emb = rows.reshape(SEQ, BATCH, EMBED_DIM)

    out_tbu = _gru_scan(
        emb,
        gru_init_state,
        kernel,
        recurrent_kernel,
        bias_input.reshape(1, 3 * UNITS),
        bias_recurrent.reshape(1, 3 * UNITS),
    )
    output = jnp.transpose(out_tbu, (1, 0, 2))
    state = out_tbu[SEQ - 1]
    return (output, state)
